# Initial kernel scaffold; baseline (speedup 1.0000x reference)
import jax
import jax.numpy as jnp
from jax.experimental import pallas as pl


def kernel(x_stock, x_other, x_connect, x_financing, x_selling, params, ei_stock_spearman_stock, ei_connect_invest_stock, ei_financing_invest_stock, ei_selling_invest_stock, ei_stock_relationship_stock, ei_stock_relationship_other, ei_other_relationship_stock, ei_other_relationship_other):
    raise NotImplementedError("write your pallas kernel here")


# speedup vs baseline: 26.4920x; 26.4920x over previous
import functools

import jax
import jax.numpy as jnp
import numpy as np
from jax import lax
from jax.experimental import pallas as pl
from jax.experimental.pallas import tpu as pltpu
from jax.experimental.pallas import tpu_sc as plsc

H = 32
OUT_DIM = 64
DH = 2
NC, NS, L = 2, 16, 16
NW = NC * NS
CH = 128

_PERM = np.array([(g % 32) * 2 + g // 32 for g in range(OUT_DIM)])


def _round_up(x, m):
    return (x + m - 1) // m * m


def _blockdiag(a):
    return jnp.einsum('hde,hg->hdge', a, jnp.eye(H, dtype=a.dtype)).reshape(OUT_DIM, OUT_DIM)


def _mm_body(x_ref, w_ref, b_ref, o_ref):
    o_ref[...] = jnp.dot(x_ref[...], w_ref[...],
                         preferred_element_type=jnp.float32) + b_ref[...]


@functools.lru_cache(maxsize=None)
def _make_mm(n, din, f, blk=512):
    return pl.pallas_call(
        _mm_body,
        grid=(pl.cdiv(n, blk),),
        in_specs=[
            pl.BlockSpec((blk, din), lambda i: (i, 0)),
            pl.BlockSpec((din, f), lambda i: (0, 0)),
            pl.BlockSpec((1, f), lambda i: (0, 0)),
        ],
        out_specs=pl.BlockSpec((blk, f), lambda i: (i, 0)),
        out_shape=jax.ShapeDtypeStruct((n, f), jnp.float32),
    )


def _matmul(x, w, b):
    n, din = x.shape
    f = w.shape[1]
    return _make_mm(n, din, f)(x, w, b.reshape(1, f))


def _out_body_skip(u0l, u1l, u0h, u1h, wa_ref, ba_ref, x_ref, beta_ref, o_ref):
    aggp = jnp.concatenate([u0l[...] + u1l[...], u0h[...] + u1h[...]], axis=1)
    a = jax.nn.gelu(aggp)
    o = jnp.dot(a, wa_ref[...], preferred_element_type=jnp.float32) + ba_ref[...]
    beta = beta_ref[0]
    o = beta * o + (1.0 - beta) * x_ref[...]
    o_ref[...] = jnp.where(o > 0, o, 0.01 * o)


def _out_body_noskip(u0l, u1l, u0h, u1h, wa_ref, ba_ref, o_ref):
    aggp = jnp.concatenate([u0l[...] + u1l[...], u0h[...] + u1h[...]], axis=1)
    a = jax.nn.gelu(aggp)
    o = jnp.dot(a, wa_ref[...], preferred_element_type=jnp.float32) + ba_ref[...]
    o_ref[...] = jnp.where(o > 0, o, 0.01 * o)


@functools.lru_cache(maxsize=None)
def _make_out_stage(n_d, has_skip, blk=512):
    specs = [pl.BlockSpec((blk, 32), lambda i: (i, 0)) for _ in range(4)]
    specs += [
        pl.BlockSpec((OUT_DIM, OUT_DIM), lambda i: (0, 0)),
        pl.BlockSpec((1, OUT_DIM), lambda i: (0, 0)),
    ]
    if has_skip:
        specs += [
            pl.BlockSpec((blk, OUT_DIM), lambda i: (i, 0)),
            pl.BlockSpec(memory_space=pltpu.SMEM),
        ]
    return pl.pallas_call(
        _out_body_skip if has_skip else _out_body_noskip,
        grid=(pl.cdiv(n_d, blk),),
        in_specs=specs,
        out_specs=pl.BlockSpec((blk, OUT_DIM), lambda i: (i, 0)),
        out_shape=jax.ShapeDtypeStruct((n_d, OUT_DIM), jnp.float32),
    )


def _head_body(x_ref, w1_ref, b1_ref, w2_ref, b2_ref, o_ref):
    a = jnp.dot(x_ref[...], w1_ref[...], preferred_element_type=jnp.float32) + b1_ref[...]
    a = jnp.where(a > 0, a, 0.01 * a)
    o_ref[...] = jnp.dot(a, w2_ref[...], preferred_element_type=jnp.float32) + b2_ref[...]


@functools.lru_cache(maxsize=None)
def _make_head(n, blk=512):
    return pl.pallas_call(
        _head_body,
        grid=(pl.cdiv(n, blk),),
        in_specs=[
            pl.BlockSpec((blk, OUT_DIM), lambda i: (i, 0)),
            pl.BlockSpec((OUT_DIM, OUT_DIM), lambda i: (0, 0)),
            pl.BlockSpec((1, OUT_DIM), lambda i: (0, 0)),
            pl.BlockSpec((OUT_DIM, 128), lambda i: (0, 0)),
            pl.BlockSpec((1, 128), lambda i: (0, 0)),
        ],
        out_specs=pl.BlockSpec((blk, 128), lambda i: (i, 0)),
        out_shape=jax.ShapeDtypeStruct((n, 128), jnp.float32),
    )


def _mesh():
    return plsc.VectorSubcoreMesh(core_axis_name="c", subcore_axis_name="s",
                                  num_cores=NC)


@functools.lru_cache(maxsize=None)
def _make_pass_a(e_pad, n_src, n_acc):
    e_per_tile = e_pad // NW
    n_chunks = e_per_tile // CH
    rows_per_tile = n_acc // NS

    @functools.partial(
        pl.kernel, mesh=_mesh(),
        compiler_params=pltpu.CompilerParams(use_tc_tiling_on_sc=False),
        out_type=[
            jax.ShapeDtypeStruct((e_pad, 32), jnp.float32),
            jax.ShapeDtypeStruct((n_acc, 32), jnp.float32),
            jax.ShapeDtypeStruct((n_acc, 32), jnp.float32),
        ],
        scratch_types=[
            pltpu.VMEM((CH,), jnp.int32),
            pltpu.VMEM((CH,), jnp.int32),
            pltpu.VMEM((CH, OUT_DIM), jnp.float32),
            pltpu.VMEM((CH, OUT_DIM), jnp.float32),
            pltpu.VMEM((CH, 32), jnp.float32),
            pltpu.VMEM_SHARED((n_acc, 32), jnp.float32),
            pltpu.SemaphoreType.DMA,
        ],
    )
    def kern(k_hbm, q_hbm, src_hbm, dst_hbm, z_hbm, ex_hbm, s0_hbm, s1_hbm,
             srcv, dstv, krows, qrows, exv, sacc, sem):
        c = lax.axis_index("c")
        s = lax.axis_index("s")
        wid = s * NC + c
        r0 = s * rows_per_tile
        pltpu.sync_copy(z_hbm.at[pl.ds(r0, rows_per_tile)],
                        sacc.at[pl.ds(r0, rows_per_tile)])
        plsc.subcore_barrier()
        base0 = wid * e_per_tile

        def chunk(i, carry):
            base = base0 + i * CH
            pltpu.sync_copy(src_hbm.at[pl.ds(base, CH)], srcv)
            pltpu.sync_copy(dst_hbm.at[pl.ds(base, CH)], dstv)
            pltpu.async_copy(k_hbm.at[srcv], krows, sem).wait()
            pltpu.async_copy(q_hbm.at[dstv], qrows, sem).wait()

            def lane(j, carry2):
                l0 = (qrows[j, pl.ds(0, 16)] * krows[j, pl.ds(0, 16)]
                      + qrows[j, pl.ds(32, 16)] * krows[j, pl.ds(32, 16)])
                l1 = (qrows[j, pl.ds(16, 16)] * krows[j, pl.ds(16, 16)]
                      + qrows[j, pl.ds(48, 16)] * krows[j, pl.ds(48, 16)])
                exv[j, pl.ds(0, 16)] = jnp.exp(l0)
                exv[j, pl.ds(16, 16)] = jnp.exp(l1)
                return carry2

            lax.fori_loop(0, CH, lane, 0)
            pltpu.sync_copy(exv, ex_hbm.at[pl.ds(base, CH)])
            pltpu.sync_copy(exv, sacc.at[dstv], add=True)
            return carry

        lax.fori_loop(0, n_chunks, chunk, 0)
        plsc.subcore_barrier()

        @pl.when(c == 0)
        def _():
            pltpu.sync_copy(sacc.at[pl.ds(r0, rows_per_tile)],
                            s0_hbm.at[pl.ds(r0, rows_per_tile)])

        @pl.when(c == 1)
        def _():
            pltpu.sync_copy(sacc.at[pl.ds(r0, rows_per_tile)],
                            s1_hbm.at[pl.ds(r0, rows_per_tile)])

    return kern


@functools.lru_cache(maxsize=None)
def _make_pass_b_lo(et_sizes, n_acc):
    n_et = len(et_sizes)
    rows_per_tile = n_acc // NS
    out_type = [jax.ShapeDtypeStruct((ep, 32), jnp.float32) for ep, _ in et_sizes]
    out_type += [jax.ShapeDtypeStruct((n_acc, 32), jnp.float32)] * 2

    @functools.partial(
        pl.kernel, mesh=_mesh(),
        compiler_params=pltpu.CompilerParams(use_tc_tiling_on_sc=False),
        out_type=out_type,
        scratch_types=[
            pltpu.VMEM((CH,), jnp.int32),
            pltpu.VMEM((CH,), jnp.int32),
            pltpu.VMEM((CH, 32), jnp.float32),
            pltpu.VMEM((CH, 32), jnp.float32),
            pltpu.VMEM((CH, 32), jnp.float32),
            pltpu.VMEM((CH, 32), jnp.float32),
            pltpu.VMEM_SHARED((n_acc, 32), jnp.float32),
            pltpu.SemaphoreType.DMA,
        ],
    )
    def kern(*refs):
        z_hbm = refs[6 * n_et]
        u0_hbm = refs[6 * n_et + 1 + n_et]
        u1_hbm = refs[6 * n_et + 2 + n_et]
        srcv, dstv, vrows, exv, s0r, s1r, uacc, sem = refs[6 * n_et + 3 + n_et:]
        c = lax.axis_index("c")
        s = lax.axis_index("s")
        wid = s * NC + c
        r0 = s * rows_per_tile
        pltpu.sync_copy(z_hbm.at[pl.ds(r0, rows_per_tile)],
                        uacc.at[pl.ds(r0, rows_per_tile)])
        plsc.subcore_barrier()

        for t in range(n_et):
            v_hbm, ex_hbm, s0_hbm, s1_hbm, src_hbm, dst_hbm = refs[6 * t:6 * t + 6]
            w_hbm = refs[6 * n_et + 1 + t]
            e_per_tile = et_sizes[t][0] // NW
            n_chunks = e_per_tile // CH
            base0 = wid * e_per_tile

            def chunk(i, carry, v_hbm=v_hbm, ex_hbm=ex_hbm, s0_hbm=s0_hbm,
                      s1_hbm=s1_hbm, src_hbm=src_hbm, dst_hbm=dst_hbm,
                      w_hbm=w_hbm, base0=base0):
                base = base0 + i * CH
                pltpu.sync_copy(src_hbm.at[pl.ds(base, CH)], srcv)
                pltpu.sync_copy(dst_hbm.at[pl.ds(base, CH)], dstv)
                pltpu.async_copy(v_hbm.at[srcv], vrows, sem).wait()
                pltpu.async_copy(s0_hbm.at[dstv], s0r, sem).wait()
                pltpu.async_copy(s1_hbm.at[dstv], s1r, sem).wait()
                pltpu.sync_copy(ex_hbm.at[pl.ds(base, CH)], exv)

                def lane(j, carry2):
                    w0 = exv[j, pl.ds(0, 16)] / (
                        s0r[j, pl.ds(0, 16)] + s1r[j, pl.ds(0, 16)] + 1e-9)
                    w1 = exv[j, pl.ds(16, 16)] / (
                        s0r[j, pl.ds(16, 16)] + s1r[j, pl.ds(16, 16)] + 1e-9)
                    exv[j, pl.ds(0, 16)] = w0
                    exv[j, pl.ds(16, 16)] = w1
                    vrows[j, pl.ds(0, 16)] = vrows[j, pl.ds(0, 16)] * w0
                    vrows[j, pl.ds(16, 16)] = vrows[j, pl.ds(16, 16)] * w1
                    return carry2

                lax.fori_loop(0, CH, lane, 0)
                pltpu.sync_copy(exv, w_hbm.at[pl.ds(base, CH)])
                pltpu.sync_copy(vrows, uacc.at[dstv], add=True)
                return carry

            lax.fori_loop(0, n_chunks, chunk, 0)

        plsc.subcore_barrier()

        @pl.when(c == 0)
        def _():
            pltpu.sync_copy(uacc.at[pl.ds(r0, rows_per_tile)],
                            u0_hbm.at[pl.ds(r0, rows_per_tile)])

        @pl.when(c == 1)
        def _():
            pltpu.sync_copy(uacc.at[pl.ds(r0, rows_per_tile)],
                            u1_hbm.at[pl.ds(r0, rows_per_tile)])

    return kern


@functools.lru_cache(maxsize=None)
def _make_pass_b_hi(et_sizes, n_acc):
    n_et = len(et_sizes)
    rows_per_tile = n_acc // NS

    @functools.partial(
        pl.kernel, mesh=_mesh(),
        compiler_params=pltpu.CompilerParams(use_tc_tiling_on_sc=False),
        out_type=[jax.ShapeDtypeStruct((n_acc, 32), jnp.float32)] * 2,
        scratch_types=[
            pltpu.VMEM((CH,), jnp.int32),
            pltpu.VMEM((CH,), jnp.int32),
            pltpu.VMEM((CH, 32), jnp.float32),
            pltpu.VMEM((CH, 32), jnp.float32),
            pltpu.VMEM_SHARED((n_acc, 32), jnp.float32),
            pltpu.SemaphoreType.DMA,
        ],
    )
    def kern(*refs):
        z_hbm = refs[4 * n_et]
        u0_hbm = refs[4 * n_et + 1]
        u1_hbm = refs[4 * n_et + 2]
        srcv, dstv, vrows, wv, uacc, sem = refs[4 * n_et + 3:]
        c = lax.axis_index("c")
        s = lax.axis_index("s")
        wid = s * NC + c
        r0 = s * rows_per_tile
        pltpu.sync_copy(z_hbm.at[pl.ds(r0, rows_per_tile)],
                        uacc.at[pl.ds(r0, rows_per_tile)])
        plsc.subcore_barrier()

        for t in range(n_et):
            v_hbm, w_hbm, src_hbm, dst_hbm = refs[4 * t:4 * t + 4]
            e_per_tile = et_sizes[t][0] // NW
            n_chunks = e_per_tile // CH
            base0 = wid * e_per_tile

            def chunk(i, carry, v_hbm=v_hbm, w_hbm=w_hbm, src_hbm=src_hbm,
                      dst_hbm=dst_hbm, base0=base0):
                base = base0 + i * CH
                pltpu.sync_copy(src_hbm.at[pl.ds(base, CH)], srcv)
                pltpu.sync_copy(dst_hbm.at[pl.ds(base, CH)], dstv)
                pltpu.async_copy(v_hbm.at[srcv], vrows, sem).wait()
                pltpu.sync_copy(w_hbm.at[pl.ds(base, CH)], wv)

                def lane(j, carry2):
                    vrows[j, pl.ds(0, 16)] = vrows[j, pl.ds(0, 16)] * wv[j, pl.ds(0, 16)]
                    vrows[j, pl.ds(16, 16)] = vrows[j, pl.ds(16, 16)] * wv[j, pl.ds(16, 16)]
                    return carry2

                lax.fori_loop(0, CH, lane, 0)
                pltpu.sync_copy(vrows, uacc.at[dstv], add=True)
                return carry

            lax.fori_loop(0, n_chunks, chunk, 0)

        plsc.subcore_barrier()

        @pl.when(c == 0)
        def _():
            pltpu.sync_copy(uacc.at[pl.ds(r0, rows_per_tile)],
                            u0_hbm.at[pl.ds(r0, rows_per_tile)])

        @pl.when(c == 1)
        def _():
            pltpu.sync_copy(uacc.at[pl.ds(r0, rows_per_tile)],
                            u1_hbm.at[pl.ds(r0, rows_per_tile)])

    return kern


def _fold_weights(pn, ep_list, is_dst):
    cols, bias = [], []
    if is_dst:
        cols.append(pn['Wq'][:, _PERM])
        bias.append(pn['bq'][_PERM])
    for ep in ep_list:
        s64 = jnp.repeat(ep['p'], DH) / np.sqrt(DH)
        ak = _blockdiag(ep['a']) * s64[None, :]
        mk = _blockdiag(ep['m'])
        cols.append((pn['Wk'] @ ak)[:, _PERM])
        bias.append((pn['bk'] @ ak)[_PERM])
        cols.append((pn['Wv'] @ mk)[:, _PERM])
        bias.append((pn['bv'] @ mk)[_PERM])
    return jnp.concatenate(cols, axis=1), jnp.concatenate(bias)


def _run_layer(x_dict, pp, ets, dst_types, edges, n_nodes, zeros_d):
    n_acc = {t: _round_up(n_nodes[t] + 1, NS * 8) for t in dst_types}
    src_ets = {t: [et for et in ets if et[0] == t] for t in x_dict}

    q, ktab, vlo, vhi = {}, {}, {}, {}
    for t, x in x_dict.items():
        is_dst = t in dst_types
        if not is_dst and not src_ets[t]:
            continue
        w, b = _fold_weights(pp['nodes'][t], [pp['edges']['__'.join(et)] for et in src_ets[t]], is_dst)
        feats = _matmul(x, w, b)
        off = 0
        if is_dst:
            q[t] = jnp.pad(feats[:, :OUT_DIM], ((0, n_acc[t] - x.shape[0]), (0, 0)))
            off = OUT_DIM
        for et in src_ets[t]:
            ktab[et] = feats[:, off:off + 64]
            vlo[et] = feats[:, off + 64:off + 96]
            vhi[et] = feats[:, off + 96:off + 128]
            off += 128

    ex, s0, s1 = {}, {}, {}
    for et in ets:
        s_t, _, d_t = et
        src_p, dst_p = edges[et]
        e_pad = src_p.shape[0]
        fn = _make_pass_a(e_pad, x_dict[s_t].shape[0], n_acc[d_t])
        ex[et], s0[et], s1[et] = fn(ktab[et], q[d_t], src_p, dst_p, zeros_d[d_t])

    h = {}
    for d_t in dst_types:
        d_ets = [et for et in ets if et[2] == d_t]
        sizes = tuple((edges[et][0].shape[0], x_dict[et[0]].shape[0]) for et in d_ets)
        args_lo = []
        for et in d_ets:
            args_lo += [vlo[et], ex[et], s0[et], s1[et], edges[et][0], edges[et][1]]
        outs = _make_pass_b_lo(sizes, n_acc[d_t])(*args_lo, zeros_d[d_t])
        ws = outs[:len(d_ets)]
        u0l, u1l = outs[len(d_ets)], outs[len(d_ets) + 1]
        args_hi = []
        for et, w_et in zip(d_ets, ws):
            args_hi += [vhi[et], w_et, edges[et][0], edges[et][1]]
        u0h, u1h = _make_pass_b_hi(sizes, n_acc[d_t])(*args_hi, zeros_d[d_t])

        n_d = n_nodes[d_t]
        pn = pp['nodes'][d_t]
        wa = pn['Wa'][_PERM, :]
        has_skip = x_dict[d_t].shape[1] == OUT_DIM
        args = [u0l, u1l, u0h, u1h, wa, pn['ba'].reshape(1, OUT_DIM)]
        if has_skip:
            beta = jax.nn.sigmoid(pn['skip']).reshape(1)
            args += [x_dict[d_t], beta]
        h[d_t] = _make_out_stage(n_d, has_skip)(*args)
    return h


def kernel(x_stock, x_other, x_connect, x_financing, x_selling, params,
           ei_stock_spearman_stock, ei_connect_invest_stock,
           ei_financing_invest_stock, ei_selling_invest_stock,
           ei_stock_relationship_stock, ei_stock_relationship_other,
           ei_other_relationship_stock, ei_other_relationship_other):
    et1 = [('stock', 'spearman', 'stock'), ('connect', 'invest', 'stock'),
           ('financing', 'invest', 'stock'), ('selling', 'invest', 'stock'),
           ('stock', 'relationship', 'stock'), ('stock', 'relationship', 'other'),
           ('other', 'relationship', 'stock'), ('other', 'relationship', 'other')]
    et2 = [et1[0], et1[4], et1[5], et1[6], et1[7]]
    et3 = [et1[0], et1[4], et1[6]]
    eis = [ei_stock_spearman_stock, ei_connect_invest_stock,
           ei_financing_invest_stock, ei_selling_invest_stock,
           ei_stock_relationship_stock, ei_stock_relationship_other,
           ei_other_relationship_stock, ei_other_relationship_other]
    n_nodes = {'stock': x_stock.shape[0], 'other': x_other.shape[0],
               'connect': x_connect.shape[0], 'financing': x_financing.shape[0],
               'selling': x_selling.shape[0]}
    n_acc = {t: _round_up(n_nodes[t] + 1, NS * 8) for t in ('stock', 'other')}
    zeros_d = {t: jnp.zeros((n_acc[t], 32), jnp.float32) for t in ('stock', 'other')}

    edges = {}
    for et, ei in zip(et1, eis):
        e = ei.shape[1]
        e_pad = _round_up(e, NW * CH)
        dummy = n_acc[et[2]] - 1
        src = jnp.concatenate([ei[0].astype(jnp.int32),
                               jnp.zeros((e_pad - e,), jnp.int32)])
        dst = jnp.concatenate([ei[1].astype(jnp.int32),
                               jnp.full((e_pad - e,), dummy, jnp.int32)])
        edges[et] = (src, dst)

    x1 = {'stock': x_stock, 'other': x_other, 'connect': x_connect,
          'financing': x_financing, 'selling': x_selling}
    h1 = _run_layer(x1, params['conv1'], et1, ('stock', 'other'), edges, n_nodes, zeros_d)
    h2 = _run_layer(h1, params['conv2'], et2, ('stock', 'other'), edges, n_nodes, zeros_d)
    h3 = _run_layer(h2, params['conv3'], et3, ('stock',), edges, n_nodes, zeros_d)

    x_sub = h3['stock'][0::12]
    w2 = jnp.pad(params['out2']['W'], ((0, 0), (0, 127)))
    b2 = jnp.pad(params['out2']['b'], (0, 127)).reshape(1, 128)
    out = _make_head(x_sub.shape[0])(x_sub, params['out1']['W'],
                                     params['out1']['b'].reshape(1, OUT_DIM), w2, b2)
    return out[:, :1]


# speedup vs baseline: 33.7670x; 1.2746x over previous
import functools

import jax
import jax.numpy as jnp
import numpy as np
from jax import lax
from jax.experimental import pallas as pl
from jax.experimental.pallas import tpu as pltpu
from jax.experimental.pallas import tpu_sc as plsc

H = 32
OUT_DIM = 64
DH = 2
NC, NS, L = 2, 16, 16
NW = NC * NS
CH = 64

_PERM = np.array([(g % 32) * 2 + g // 32 for g in range(OUT_DIM)])


def _round_up(x, m):
    return (x + m - 1) // m * m


def _blockdiag(a):
    return jnp.einsum('hde,hg->hdge', a, jnp.eye(H, dtype=a.dtype)).reshape(OUT_DIM, OUT_DIM)


def _mm_body(x_ref, w_ref, b_ref, o_ref):
    o_ref[...] = jnp.dot(x_ref[...], w_ref[...],
                         preferred_element_type=jnp.float32) + b_ref[...]


@functools.lru_cache(maxsize=None)
def _make_mm(n, din, f, blk=512):
    return pl.pallas_call(
        _mm_body,
        grid=(pl.cdiv(n, blk),),
        in_specs=[
            pl.BlockSpec((blk, din), lambda i: (i, 0)),
            pl.BlockSpec((din, f), lambda i: (0, 0)),
            pl.BlockSpec((1, f), lambda i: (0, 0)),
        ],
        out_specs=pl.BlockSpec((blk, f), lambda i: (i, 0)),
        out_shape=jax.ShapeDtypeStruct((n, f), jnp.float32),
    )


def _matmul(x, w, b):
    n, din = x.shape
    f = w.shape[1]
    return _make_mm(n, din, f)(x, w, b.reshape(1, f))


def _out_body_skip(u0l, u1l, u0h, u1h, wa_ref, ba_ref, x_ref, beta_ref, o_ref):
    aggp = jnp.concatenate([u0l[...] + u1l[...], u0h[...] + u1h[...]], axis=1)
    a = jax.nn.gelu(aggp)
    o = jnp.dot(a, wa_ref[...], preferred_element_type=jnp.float32) + ba_ref[...]
    beta = beta_ref[0]
    o = beta * o + (1.0 - beta) * x_ref[...]
    o_ref[...] = jnp.where(o > 0, o, 0.01 * o)


def _out_body_noskip(u0l, u1l, u0h, u1h, wa_ref, ba_ref, o_ref):
    aggp = jnp.concatenate([u0l[...] + u1l[...], u0h[...] + u1h[...]], axis=1)
    a = jax.nn.gelu(aggp)
    o = jnp.dot(a, wa_ref[...], preferred_element_type=jnp.float32) + ba_ref[...]
    o_ref[...] = jnp.where(o > 0, o, 0.01 * o)


@functools.lru_cache(maxsize=None)
def _make_out_stage(n_d, has_skip, blk=512):
    specs = [pl.BlockSpec((blk, 32), lambda i: (i, 0)) for _ in range(4)]
    specs += [
        pl.BlockSpec((OUT_DIM, OUT_DIM), lambda i: (0, 0)),
        pl.BlockSpec((1, OUT_DIM), lambda i: (0, 0)),
    ]
    if has_skip:
        specs += [
            pl.BlockSpec((blk, OUT_DIM), lambda i: (i, 0)),
            pl.BlockSpec(memory_space=pltpu.SMEM),
        ]
    return pl.pallas_call(
        _out_body_skip if has_skip else _out_body_noskip,
        grid=(pl.cdiv(n_d, blk),),
        in_specs=specs,
        out_specs=pl.BlockSpec((blk, OUT_DIM), lambda i: (i, 0)),
        out_shape=jax.ShapeDtypeStruct((n_d, OUT_DIM), jnp.float32),
    )


def _head_body(x_ref, w1_ref, b1_ref, w2_ref, b2_ref, o_ref):
    a = jnp.dot(x_ref[...], w1_ref[...], preferred_element_type=jnp.float32) + b1_ref[...]
    a = jnp.where(a > 0, a, 0.01 * a)
    o_ref[...] = jnp.dot(a, w2_ref[...], preferred_element_type=jnp.float32) + b2_ref[...]


@functools.lru_cache(maxsize=None)
def _make_head(n, blk=512):
    return pl.pallas_call(
        _head_body,
        grid=(pl.cdiv(n, blk),),
        in_specs=[
            pl.BlockSpec((blk, OUT_DIM), lambda i: (i, 0)),
            pl.BlockSpec((OUT_DIM, OUT_DIM), lambda i: (0, 0)),
            pl.BlockSpec((1, OUT_DIM), lambda i: (0, 0)),
            pl.BlockSpec((OUT_DIM, 128), lambda i: (0, 0)),
            pl.BlockSpec((1, 128), lambda i: (0, 0)),
        ],
        out_specs=pl.BlockSpec((blk, 128), lambda i: (i, 0)),
        out_shape=jax.ShapeDtypeStruct((n, 128), jnp.float32),
    )


def _mesh():
    return plsc.VectorSubcoreMesh(core_axis_name="c", subcore_axis_name="s",
                                  num_cores=NC)


@functools.lru_cache(maxsize=None)
def _make_pass_a(e_pad, n_src, n_acc):
    e_per_tile = e_pad // NW
    n_pairs = e_per_tile // (2 * CH)
    rows_per_tile = n_acc // NS

    @functools.partial(
        pl.kernel, mesh=_mesh(),
        compiler_params=pltpu.CompilerParams(use_tc_tiling_on_sc=False),
        out_type=[
            jax.ShapeDtypeStruct((e_pad, 32), jnp.float32),
            jax.ShapeDtypeStruct((n_acc, 32), jnp.float32),
            jax.ShapeDtypeStruct((n_acc, 32), jnp.float32),
        ],
        scratch_types=[
            pltpu.VMEM((CH,), jnp.int32),
            pltpu.VMEM((CH,), jnp.int32),
            pltpu.VMEM((CH, OUT_DIM), jnp.float32),
            pltpu.VMEM((CH, OUT_DIM), jnp.float32),
            pltpu.VMEM((CH,), jnp.int32),
            pltpu.VMEM((CH,), jnp.int32),
            pltpu.VMEM((CH, OUT_DIM), jnp.float32),
            pltpu.VMEM((CH, OUT_DIM), jnp.float32),
            pltpu.VMEM((CH, 32), jnp.float32),
            pltpu.VMEM_SHARED((n_acc, 32), jnp.float32),
            pltpu.SemaphoreType.DMA,
            pltpu.SemaphoreType.DMA,
        ],
    )
    def kern(k_hbm, q_hbm, src_hbm, dst_hbm, z_hbm, ex_hbm, s0_hbm, s1_hbm,
             srcv0, dstv0, krows0, qrows0, srcv1, dstv1, krows1, qrows1,
             exv, sacc, sem0, sem1):
        srcv, dstv = [srcv0, srcv1], [dstv0, dstv1]
        krows, qrows = [krows0, krows1], [qrows0, qrows1]
        sem = [sem0, sem1]
        c = lax.axis_index("c")
        s = lax.axis_index("s")
        wid = s * NC + c
        r0 = s * rows_per_tile
        pltpu.sync_copy(z_hbm.at[pl.ds(r0, rows_per_tile)],
                        sacc.at[pl.ds(r0, rows_per_tile)])
        plsc.subcore_barrier()
        base0 = wid * e_per_tile

        def prefetch(b, base):
            pltpu.sync_copy(src_hbm.at[pl.ds(base, CH)], srcv[b])
            pltpu.sync_copy(dst_hbm.at[pl.ds(base, CH)], dstv[b])
            pltpu.async_copy(k_hbm.at[srcv[b]], krows[b], sem[b])
            pltpu.async_copy(q_hbm.at[dstv[b]], qrows[b], sem[b])

        def waitb(b):
            pltpu.make_async_copy(k_hbm.at[srcv[b]], krows[b], sem[b]).wait()
            pltpu.make_async_copy(q_hbm.at[dstv[b]], qrows[b], sem[b]).wait()

        def compute(b, base):
            def lane(j, carry2):
                l0 = (qrows[b][j, pl.ds(0, 16)] * krows[b][j, pl.ds(0, 16)]
                      + qrows[b][j, pl.ds(32, 16)] * krows[b][j, pl.ds(32, 16)])
                l1 = (qrows[b][j, pl.ds(16, 16)] * krows[b][j, pl.ds(16, 16)]
                      + qrows[b][j, pl.ds(48, 16)] * krows[b][j, pl.ds(48, 16)])
                exv[j, pl.ds(0, 16)] = jnp.exp(l0)
                exv[j, pl.ds(16, 16)] = jnp.exp(l1)
                return carry2

            lax.fori_loop(0, CH, lane, 0)
            pltpu.sync_copy(exv, ex_hbm.at[pl.ds(base, CH)])
            pltpu.sync_copy(exv, sacc.at[dstv[b]], add=True)

        prefetch(0, base0)

        def pair(i, carry):
            base = base0 + i * (2 * CH)
            waitb(0)
            prefetch(1, base + CH)
            compute(0, base)
            waitb(1)

            @pl.when(i + 1 < n_pairs)
            def _():
                prefetch(0, base + 2 * CH)

            compute(1, base + CH)
            return carry

        lax.fori_loop(0, n_pairs, pair, 0)
        plsc.subcore_barrier()

        @pl.when(c == 0)
        def _():
            pltpu.sync_copy(sacc.at[pl.ds(r0, rows_per_tile)],
                            s0_hbm.at[pl.ds(r0, rows_per_tile)])

        @pl.when(c == 1)
        def _():
            pltpu.sync_copy(sacc.at[pl.ds(r0, rows_per_tile)],
                            s1_hbm.at[pl.ds(r0, rows_per_tile)])

    return kern


@functools.lru_cache(maxsize=None)
def _make_pass_b_lo(et_sizes, n_acc):
    n_et = len(et_sizes)
    rows_per_tile = n_acc // NS
    out_type = [jax.ShapeDtypeStruct((ep, 32), jnp.float32) for ep, _ in et_sizes]
    out_type += [jax.ShapeDtypeStruct((n_acc, 32), jnp.float32)] * 2

    @functools.partial(
        pl.kernel, mesh=_mesh(),
        compiler_params=pltpu.CompilerParams(use_tc_tiling_on_sc=False),
        out_type=out_type,
        scratch_types=[
            pltpu.VMEM((CH,), jnp.int32),
            pltpu.VMEM((CH,), jnp.int32),
            pltpu.VMEM((CH, 32), jnp.float32),
            pltpu.VMEM((CH, 32), jnp.float32),
            pltpu.VMEM((CH, 32), jnp.float32),
            pltpu.VMEM((CH, 32), jnp.float32),
            pltpu.VMEM((CH,), jnp.int32),
            pltpu.VMEM((CH,), jnp.int32),
            pltpu.VMEM((CH, 32), jnp.float32),
            pltpu.VMEM((CH, 32), jnp.float32),
            pltpu.VMEM((CH, 32), jnp.float32),
            pltpu.VMEM((CH, 32), jnp.float32),
            pltpu.VMEM_SHARED((n_acc, 32), jnp.float32),
            pltpu.SemaphoreType.DMA,
            pltpu.SemaphoreType.DMA,
        ],
    )
    def kern(*refs):
        z_hbm = refs[6 * n_et]
        u0_hbm = refs[6 * n_et + 1 + n_et]
        u1_hbm = refs[6 * n_et + 2 + n_et]
        (srcv0, dstv0, vrows0, exv0, s0r0, s1r0,
         srcv1, dstv1, vrows1, exv1, s0r1, s1r1,
         uacc, sem0, sem1) = refs[6 * n_et + 3 + n_et:]
        srcv, dstv = [srcv0, srcv1], [dstv0, dstv1]
        vrows, exv = [vrows0, vrows1], [exv0, exv1]
        s0r, s1r = [s0r0, s0r1], [s1r0, s1r1]
        sem = [sem0, sem1]
        c = lax.axis_index("c")
        s = lax.axis_index("s")
        wid = s * NC + c
        r0 = s * rows_per_tile
        pltpu.sync_copy(z_hbm.at[pl.ds(r0, rows_per_tile)],
                        uacc.at[pl.ds(r0, rows_per_tile)])
        plsc.subcore_barrier()

        for t in range(n_et):
            v_hbm, ex_hbm, s0_hbm, s1_hbm, src_hbm, dst_hbm = refs[6 * t:6 * t + 6]
            w_hbm = refs[6 * n_et + 1 + t]
            e_per_tile = et_sizes[t][0] // NW
            n_pairs = e_per_tile // (2 * CH)
            base0 = wid * e_per_tile

            def prefetch(b, base, v_hbm=v_hbm, ex_hbm=ex_hbm, s0_hbm=s0_hbm,
                         s1_hbm=s1_hbm, src_hbm=src_hbm, dst_hbm=dst_hbm):
                pltpu.sync_copy(src_hbm.at[pl.ds(base, CH)], srcv[b])
                pltpu.sync_copy(dst_hbm.at[pl.ds(base, CH)], dstv[b])
                pltpu.async_copy(v_hbm.at[srcv[b]], vrows[b], sem[b])
                pltpu.async_copy(s0_hbm.at[dstv[b]], s0r[b], sem[b])
                pltpu.async_copy(s1_hbm.at[dstv[b]], s1r[b], sem[b])
                pltpu.async_copy(ex_hbm.at[pl.ds(base, CH)], exv[b], sem[b])

            def waitb(b, v_hbm=v_hbm, ex_hbm=ex_hbm, s0_hbm=s0_hbm,
                      s1_hbm=s1_hbm):
                pltpu.make_async_copy(v_hbm.at[srcv[b]], vrows[b], sem[b]).wait()
                pltpu.make_async_copy(s0_hbm.at[dstv[b]], s0r[b], sem[b]).wait()
                pltpu.make_async_copy(s1_hbm.at[dstv[b]], s1r[b], sem[b]).wait()
                pltpu.make_async_copy(ex_hbm.at[pl.ds(0, CH)], exv[b], sem[b]).wait()

            def compute(b, base, w_hbm=w_hbm):
                def lane(j, carry2):
                    w0 = exv[b][j, pl.ds(0, 16)] / (
                        s0r[b][j, pl.ds(0, 16)] + s1r[b][j, pl.ds(0, 16)] + 1e-9)
                    w1 = exv[b][j, pl.ds(16, 16)] / (
                        s0r[b][j, pl.ds(16, 16)] + s1r[b][j, pl.ds(16, 16)] + 1e-9)
                    exv[b][j, pl.ds(0, 16)] = w0
                    exv[b][j, pl.ds(16, 16)] = w1
                    vrows[b][j, pl.ds(0, 16)] = vrows[b][j, pl.ds(0, 16)] * w0
                    vrows[b][j, pl.ds(16, 16)] = vrows[b][j, pl.ds(16, 16)] * w1
                    return carry2

                lax.fori_loop(0, CH, lane, 0)
                pltpu.sync_copy(exv[b], w_hbm.at[pl.ds(base, CH)])
                pltpu.sync_copy(vrows[b], uacc.at[dstv[b]], add=True)

            prefetch(0, base0)

            def pair(i, carry, base0=base0, prefetch=prefetch, waitb=waitb,
                     compute=compute, n_pairs=n_pairs):
                base = base0 + i * (2 * CH)
                waitb(0)
                prefetch(1, base + CH)
                compute(0, base)
                waitb(1)

                @pl.when(i + 1 < n_pairs)
                def _():
                    prefetch(0, base + 2 * CH)

                compute(1, base + CH)
                return carry

            lax.fori_loop(0, n_pairs, pair, 0)

        plsc.subcore_barrier()

        @pl.when(c == 0)
        def _():
            pltpu.sync_copy(uacc.at[pl.ds(r0, rows_per_tile)],
                            u0_hbm.at[pl.ds(r0, rows_per_tile)])

        @pl.when(c == 1)
        def _():
            pltpu.sync_copy(uacc.at[pl.ds(r0, rows_per_tile)],
                            u1_hbm.at[pl.ds(r0, rows_per_tile)])

    return kern


@functools.lru_cache(maxsize=None)
def _make_pass_b_hi(et_sizes, n_acc):
    n_et = len(et_sizes)
    rows_per_tile = n_acc // NS

    @functools.partial(
        pl.kernel, mesh=_mesh(),
        compiler_params=pltpu.CompilerParams(use_tc_tiling_on_sc=False),
        out_type=[jax.ShapeDtypeStruct((n_acc, 32), jnp.float32)] * 2,
        scratch_types=[
            pltpu.VMEM((CH,), jnp.int32),
            pltpu.VMEM((CH,), jnp.int32),
            pltpu.VMEM((CH, 32), jnp.float32),
            pltpu.VMEM((CH, 32), jnp.float32),
            pltpu.VMEM((CH,), jnp.int32),
            pltpu.VMEM((CH,), jnp.int32),
            pltpu.VMEM((CH, 32), jnp.float32),
            pltpu.VMEM((CH, 32), jnp.float32),
            pltpu.VMEM_SHARED((n_acc, 32), jnp.float32),
            pltpu.SemaphoreType.DMA,
            pltpu.SemaphoreType.DMA,
        ],
    )
    def kern(*refs):
        z_hbm = refs[4 * n_et]
        u0_hbm = refs[4 * n_et + 1]
        u1_hbm = refs[4 * n_et + 2]
        (srcv0, dstv0, vrows0, wv0, srcv1, dstv1, vrows1, wv1,
         uacc, sem0, sem1) = refs[4 * n_et + 3:]
        srcv, dstv = [srcv0, srcv1], [dstv0, dstv1]
        vrows, wv = [vrows0, vrows1], [wv0, wv1]
        sem = [sem0, sem1]
        c = lax.axis_index("c")
        s = lax.axis_index("s")
        wid = s * NC + c
        r0 = s * rows_per_tile
        pltpu.sync_copy(z_hbm.at[pl.ds(r0, rows_per_tile)],
                        uacc.at[pl.ds(r0, rows_per_tile)])
        plsc.subcore_barrier()

        for t in range(n_et):
            v_hbm, w_hbm, src_hbm, dst_hbm = refs[4 * t:4 * t + 4]
            e_per_tile = et_sizes[t][0] // NW
            n_pairs = e_per_tile // (2 * CH)
            base0 = wid * e_per_tile

            def prefetch(b, base, v_hbm=v_hbm, w_hbm=w_hbm, src_hbm=src_hbm,
                         dst_hbm=dst_hbm):
                pltpu.sync_copy(src_hbm.at[pl.ds(base, CH)], srcv[b])
                pltpu.sync_copy(dst_hbm.at[pl.ds(base, CH)], dstv[b])
                pltpu.async_copy(v_hbm.at[srcv[b]], vrows[b], sem[b])
                pltpu.async_copy(w_hbm.at[pl.ds(base, CH)], wv[b], sem[b])

            def waitb(b, v_hbm=v_hbm, w_hbm=w_hbm):
                pltpu.make_async_copy(v_hbm.at[srcv[b]], vrows[b], sem[b]).wait()
                pltpu.make_async_copy(w_hbm.at[pl.ds(0, CH)], wv[b], sem[b]).wait()

            def compute(b, base):
                def lane(j, carry2):
                    vrows[b][j, pl.ds(0, 16)] = (
                        vrows[b][j, pl.ds(0, 16)] * wv[b][j, pl.ds(0, 16)])
                    vrows[b][j, pl.ds(16, 16)] = (
                        vrows[b][j, pl.ds(16, 16)] * wv[b][j, pl.ds(16, 16)])
                    return carry2

                lax.fori_loop(0, CH, lane, 0)
                pltpu.sync_copy(vrows[b], uacc.at[dstv[b]], add=True)

            prefetch(0, base0)

            def pair(i, carry, base0=base0, prefetch=prefetch, waitb=waitb,
                     compute=compute, n_pairs=n_pairs):
                base = base0 + i * (2 * CH)
                waitb(0)
                prefetch(1, base + CH)
                compute(0, base)
                waitb(1)

                @pl.when(i + 1 < n_pairs)
                def _():
                    prefetch(0, base + 2 * CH)

                compute(1, base + CH)
                return carry

            lax.fori_loop(0, n_pairs, pair, 0)

        plsc.subcore_barrier()

        @pl.when(c == 0)
        def _():
            pltpu.sync_copy(uacc.at[pl.ds(r0, rows_per_tile)],
                            u0_hbm.at[pl.ds(r0, rows_per_tile)])

        @pl.when(c == 1)
        def _():
            pltpu.sync_copy(uacc.at[pl.ds(r0, rows_per_tile)],
                            u1_hbm.at[pl.ds(r0, rows_per_tile)])

    return kern


def _fold_weights(pn, ep_list, is_dst):
    cols, bias = [], []
    if is_dst:
        cols.append(pn['Wq'][:, _PERM])
        bias.append(pn['bq'][_PERM])
    for ep in ep_list:
        s64 = jnp.repeat(ep['p'], DH) / np.sqrt(DH)
        ak = _blockdiag(ep['a']) * s64[None, :]
        mk = _blockdiag(ep['m'])
        cols.append((pn['Wk'] @ ak)[:, _PERM])
        bias.append((pn['bk'] @ ak)[_PERM])
        cols.append((pn['Wv'] @ mk)[:, _PERM])
        bias.append((pn['bv'] @ mk)[_PERM])
    return jnp.concatenate(cols, axis=1), jnp.concatenate(bias)


def _run_layer(x_dict, pp, ets, dst_types, edges, n_nodes, zeros_d):
    n_acc = {t: _round_up(n_nodes[t] + 1, NS * 8) for t in dst_types}
    src_ets = {t: [et for et in ets if et[0] == t] for t in x_dict}

    q, ktab, vlo, vhi = {}, {}, {}, {}
    for t, x in x_dict.items():
        is_dst = t in dst_types
        if not is_dst and not src_ets[t]:
            continue
        w, b = _fold_weights(pp['nodes'][t], [pp['edges']['__'.join(et)] for et in src_ets[t]], is_dst)
        feats = _matmul(x, w, b)
        off = 0
        if is_dst:
            q[t] = jnp.pad(feats[:, :OUT_DIM], ((0, n_acc[t] - x.shape[0]), (0, 0)))
            off = OUT_DIM
        for et in src_ets[t]:
            ktab[et] = feats[:, off:off + 64]
            vlo[et] = feats[:, off + 64:off + 96]
            vhi[et] = feats[:, off + 96:off + 128]
            off += 128

    ex, s0, s1 = {}, {}, {}
    for et in ets:
        s_t, _, d_t = et
        src_p, dst_p = edges[et]
        e_pad = src_p.shape[0]
        fn = _make_pass_a(e_pad, x_dict[s_t].shape[0], n_acc[d_t])
        ex[et], s0[et], s1[et] = fn(ktab[et], q[d_t], src_p, dst_p, zeros_d[d_t])

    h = {}
    for d_t in dst_types:
        d_ets = [et for et in ets if et[2] == d_t]
        sizes = tuple((edges[et][0].shape[0], x_dict[et[0]].shape[0]) for et in d_ets)
        args_lo = []
        for et in d_ets:
            args_lo += [vlo[et], ex[et], s0[et], s1[et], edges[et][0], edges[et][1]]
        outs = _make_pass_b_lo(sizes, n_acc[d_t])(*args_lo, zeros_d[d_t])
        ws = outs[:len(d_ets)]
        u0l, u1l = outs[len(d_ets)], outs[len(d_ets) + 1]
        args_hi = []
        for et, w_et in zip(d_ets, ws):
            args_hi += [vhi[et], w_et, edges[et][0], edges[et][1]]
        u0h, u1h = _make_pass_b_hi(sizes, n_acc[d_t])(*args_hi, zeros_d[d_t])

        n_d = n_nodes[d_t]
        pn = pp['nodes'][d_t]
        wa = pn['Wa'][_PERM, :]
        has_skip = x_dict[d_t].shape[1] == OUT_DIM
        args = [u0l, u1l, u0h, u1h, wa, pn['ba'].reshape(1, OUT_DIM)]
        if has_skip:
            beta = jax.nn.sigmoid(pn['skip']).reshape(1)
            args += [x_dict[d_t], beta]
        h[d_t] = _make_out_stage(n_d, has_skip)(*args)
    return h


def kernel(x_stock, x_other, x_connect, x_financing, x_selling, params,
           ei_stock_spearman_stock, ei_connect_invest_stock,
           ei_financing_invest_stock, ei_selling_invest_stock,
           ei_stock_relationship_stock, ei_stock_relationship_other,
           ei_other_relationship_stock, ei_other_relationship_other):
    et1 = [('stock', 'spearman', 'stock'), ('connect', 'invest', 'stock'),
           ('financing', 'invest', 'stock'), ('selling', 'invest', 'stock'),
           ('stock', 'relationship', 'stock'), ('stock', 'relationship', 'other'),
           ('other', 'relationship', 'stock'), ('other', 'relationship', 'other')]
    et2 = [et1[0], et1[4], et1[5], et1[6], et1[7]]
    et3 = [et1[0], et1[4], et1[6]]
    eis = [ei_stock_spearman_stock, ei_connect_invest_stock,
           ei_financing_invest_stock, ei_selling_invest_stock,
           ei_stock_relationship_stock, ei_stock_relationship_other,
           ei_other_relationship_stock, ei_other_relationship_other]
    n_nodes = {'stock': x_stock.shape[0], 'other': x_other.shape[0],
               'connect': x_connect.shape[0], 'financing': x_financing.shape[0],
               'selling': x_selling.shape[0]}
    n_acc = {t: _round_up(n_nodes[t] + 1, NS * 8) for t in ('stock', 'other')}
    zeros_d = {t: jnp.zeros((n_acc[t], 32), jnp.float32) for t in ('stock', 'other')}

    edges = {}
    for et, ei in zip(et1, eis):
        e = ei.shape[1]
        e_pad = _round_up(e, NW * CH * 2)
        dummy = n_acc[et[2]] - 1
        src = jnp.concatenate([ei[0].astype(jnp.int32),
                               jnp.zeros((e_pad - e,), jnp.int32)])
        dst = jnp.concatenate([ei[1].astype(jnp.int32),
                               jnp.full((e_pad - e,), dummy, jnp.int32)])
        edges[et] = (src, dst)

    x1 = {'stock': x_stock, 'other': x_other, 'connect': x_connect,
          'financing': x_financing, 'selling': x_selling}
    h1 = _run_layer(x1, params['conv1'], et1, ('stock', 'other'), edges, n_nodes, zeros_d)
    h2 = _run_layer(h1, params['conv2'], et2, ('stock', 'other'), edges, n_nodes, zeros_d)
    h3 = _run_layer(h2, params['conv3'], et3, ('stock',), edges, n_nodes, zeros_d)

    x_sub = h3['stock'][0::12]
    w2 = jnp.pad(params['out2']['W'], ((0, 0), (0, 127)))
    b2 = jnp.pad(params['out2']['b'], (0, 127)).reshape(1, 128)
    out = _make_head(x_sub.shape[0])(x_sub, params['out1']['W'],
                                     params['out1']['b'].reshape(1, OUT_DIM), w2, b2)
    return out[:, :1]
